# baseline (speedup 1.0000x reference)
import functools

import jax
import jax.numpy as jnp
from jax import lax
from jax.experimental import pallas as pl
from jax.experimental.pallas import tpu as pltpu
from jax.experimental.pallas import tpu_sc as plsc

_D = 32
_PD = 128
_NW = 32
_CR = 8
_PM = 32


def _emb_kernel(n_rows, m, table_hbm, idx_hbm, out_hbm,
                idx_v, g_a, g_b, sem_ga, sem_gb, sem_wa, sem_wb):
    wid = lax.axis_index("s") * 2 + lax.axis_index("c")
    rpw = n_rows // _NW
    base = wid * rpw
    ipw = rpw * _PM
    pltpu.sync_copy(idx_hbm.at[pl.ds(wid * ipw, ipw)], idx_v)

    n_pairs = rpw // _CR // 2

    def g_start(buf, sem, c):
        def one(r, carry):
            pltpu.async_copy(
                table_hbm.at[idx_v.at[pl.ds((c * _CR + r) * _PM, _PM)]],
                buf.at[r], sem)
            return carry

        lax.fori_loop(0, _CR, one, 0)

    def g_drain(buf, sem):
        def one(r, carry):
            pltpu.make_async_copy(
                table_hbm.at[idx_v.at[pl.ds(0, _PM)]], buf.at[r], sem).wait()
            return carry

        lax.fori_loop(0, _CR, one, 0)

    g_start(g_a, sem_ga, 0)
    g_start(g_b, sem_gb, 1)

    def body(p, carry):
        c0 = p * 2
        g_drain(g_a, sem_ga)
        wa = pltpu.async_copy(
            g_a.at[:, pl.ds(0, m), pl.ds(0, _D)],
            out_hbm.at[pl.ds(base + c0 * _CR, _CR)], sem_wa)
        g_drain(g_b, sem_gb)
        wb = pltpu.async_copy(
            g_b.at[:, pl.ds(0, m), pl.ds(0, _D)],
            out_hbm.at[pl.ds(base + (c0 + 1) * _CR, _CR)], sem_wb)

        wa.wait()

        @pl.when(p < n_pairs - 1)
        def _():
            g_start(g_a, sem_ga, c0 + 2)

        wb.wait()

        @pl.when(p < n_pairs - 1)
        def _():
            g_start(g_b, sem_gb, c0 + 3)

        return carry

    lax.fori_loop(0, n_pairs, body, 0)


def kernel(x, emb_weight):
    n, m = x.shape
    nv, d = emb_weight.shape
    idx1d = jnp.pad(x.astype(jnp.int32), ((0, 0), (0, _PM - m))).reshape(-1)
    tab_pad = jnp.pad(emb_weight, ((0, 0), (0, _PD - d)))
    ipw = n * _PM // _NW

    mesh = plsc.VectorSubcoreMesh(core_axis_name="c", subcore_axis_name="s")

    k = functools.partial(
        pl.kernel,
        mesh=mesh,
        out_type=jax.ShapeDtypeStruct((n, m, _D), jnp.float32),
        scratch_types=[
            pltpu.VMEM((ipw,), jnp.int32),
            pltpu.VMEM((_CR, _PM, _PD), jnp.float32),
            pltpu.VMEM((_CR, _PM, _PD), jnp.float32),
            pltpu.SemaphoreType.DMA,
            pltpu.SemaphoreType.DMA,
            pltpu.SemaphoreType.DMA,
            pltpu.SemaphoreType.DMA,
        ],
        compiler_params=pltpu.CompilerParams(use_tc_tiling_on_sc=False),
    )(functools.partial(_emb_kernel, n, m))

    return k(tab_pad, idx1d)


# speedup vs baseline: 5.9290x; 5.9290x over previous
import functools

import jax
import jax.numpy as jnp
from jax import lax
from jax.experimental import pallas as pl
from jax.experimental.pallas import tpu as pltpu
from jax.experimental.pallas import tpu_sc as plsc

_D = 32
_NW = 32
_CR = 64


def _emb_kernel(n_rows, m, table_hbm, idx_hbm, out_hbm,
                idx_v, rows_a, rows_b, sem_ga, sem_gb, sem_wa, sem_wb):
    wid = lax.axis_index("s") * 2 + lax.axis_index("c")
    rpw = n_rows // _NW
    base = wid * rpw
    pltpu.sync_copy(idx_hbm.at[pl.ds(base, rpw)], idx_v)

    n_pairs = rpw // _CR // 2

    def g_start(buf, sem, c):
        def one(r, carry):
            pltpu.async_copy(
                table_hbm.at[idx_v.at[c * _CR + r]], buf.at[r], sem)
            return carry

        lax.fori_loop(0, _CR, one, 0)

    def g_drain(buf, sem):
        pltpu.make_async_copy(out_hbm.at[pl.ds(0, _CR)], buf, sem).wait()

    g_start(rows_a, sem_ga, 0)
    g_start(rows_b, sem_gb, 1)

    def body(p, carry):
        c0 = p * 2
        g_drain(rows_a, sem_ga)
        wa = pltpu.async_copy(
            rows_a, out_hbm.at[pl.ds(base + c0 * _CR, _CR)], sem_wa)
        g_drain(rows_b, sem_gb)
        wb = pltpu.async_copy(
            rows_b, out_hbm.at[pl.ds(base + (c0 + 1) * _CR, _CR)], sem_wb)

        wa.wait()

        @pl.when(p < n_pairs - 1)
        def _():
            g_start(rows_a, sem_ga, c0 + 2)

        wb.wait()

        @pl.when(p < n_pairs - 1)
        def _():
            g_start(rows_b, sem_gb, c0 + 3)

        return carry

    lax.fori_loop(0, n_pairs, body, 0)


def kernel(x, emb_weight):
    n, m = x.shape
    idx2d = x.astype(jnp.int32)
    rpw = n // _NW

    mesh = plsc.VectorSubcoreMesh(core_axis_name="c", subcore_axis_name="s")

    k = functools.partial(
        pl.kernel,
        mesh=mesh,
        out_type=jax.ShapeDtypeStruct((n, m, _D), jnp.float32),
        scratch_types=[
            pltpu.VMEM((rpw, m), jnp.int32),
            pltpu.VMEM((_CR, m, _D), jnp.float32),
            pltpu.VMEM((_CR, m, _D), jnp.float32),
            pltpu.SemaphoreType.DMA,
            pltpu.SemaphoreType.DMA,
            pltpu.SemaphoreType.DMA,
            pltpu.SemaphoreType.DMA,
        ],
        compiler_params=pltpu.CompilerParams(use_tc_tiling_on_sc=False),
    )(functools.partial(_emb_kernel, n, m))

    return k(emb_weight, idx2d)
